# baseline (speedup 1.0000x reference)
import functools

import jax
import jax.numpy as jnp
from jax import lax
from jax.experimental import pallas as pl
from jax.experimental.pallas import tpu as pltpu
from jax.experimental.pallas import tpu_sc as plsc

B = 4096
F = 26
L = 50
VOCAB = 100000
D = 32

NC = 2
NS = 16
NW = NC * NS
BPW = B // NW
CB = 32
NCHUNK = BPW // CB

FP = F + 1
SPC = CB * FP
SG = 96
NSG = SPC // SG
QPC = CB * L
QG = 100
NQG = QPC // QG
LU = 10

_mesh = plsc.VectorSubcoreMesh(core_axis_name="c", subcore_axis_name="s")


def _splat_f32(x):
    return jnp.full((16,), x, jnp.float32)


def _splat_i32(x):
    return jnp.full((16,), x, jnp.int32)


@functools.partial(
    pl.kernel,
    mesh=_mesh,
    compiler_params=pltpu.CompilerParams(use_tc_tiling_on_sc=False, needs_layout_passes=False),
    out_type=jax.ShapeDtypeStruct((B * FP, D), jnp.float32),
    scratch_types=[
        pltpu.VMEM((NCHUNK * NSG, SG), jnp.int32),
        pltpu.VMEM((NQG, QG), jnp.int32),
        pltpu.VMEM((CB * 64,), jnp.int32),
        pltpu.VMEM((SPC, D), jnp.float32),
        pltpu.VMEM((QPC, D), jnp.float32),
        pltpu.VMEM((CB * 16,), jnp.float32),
        pltpu.VMEM((432,), jnp.int32),
        pltpu.SemaphoreType.DMA,
    ],
)
def _sc_embed(ftab, qtab, sidx, qidx, pidx, off, out,
              sidx_v, qidx_v, pidx_v, srows_v, qrows_v, recip_v, off_v, sem):
    wid = lax.axis_index("s") * NC + lax.axis_index("c")

    pltpu.sync_copy(off, off_v)

    pltpu.sync_copy(sidx.at[wid], sidx_v)
    for g in range(NCHUNK * NSG):
        for k in range(SG // 16):
            o = (g * SG + k * 16) % 432
            sidx_v[g, pl.ds(k * 16, 16)] = (
                sidx_v[g, pl.ds(k * 16, 16)] + off_v[pl.ds(o, 16)]
            )

    iota16 = lax.iota(jnp.int32, 16)

    for c in range(NCHUNK):
        b0 = wid * BPW + c * CB
        pltpu.sync_copy(
            qidx.at[pl.ds(pl.multiple_of(b0 * L // QG, 16), NQG)], qidx_v)
        pltpu.sync_copy(
            pidx.at[pl.ds(pl.multiple_of(b0 * 64, 32), CB * 64)], pidx_v)

        handles = []
        for g in range(NSG):
            handles.append(pltpu.async_copy(
                ftab.at[sidx_v.at[c * NSG + g]],
                srows_v.at[pl.ds(g * SG, SG)], sem))
        for g in range(NQG):
            handles.append(pltpu.async_copy(
                qtab.at[qidx_v.at[g]], qrows_v.at[pl.ds(g * QG, QG)], sem))

        for bg in range(CB // 16):
            rows_v = iota16 + _splat_i32(bg * 16)
            rbase = rows_v * _splat_i32(64)
            m = _splat_f32(0.0)
            for l in range(L):
                v = plsc.load_gather(pidx_v, [rbase + _splat_i32(l)])
                m = m + jnp.where(v != 0, _splat_f32(1.0), _splat_f32(0.0))
            rv = _splat_f32(1.0) / (m + _splat_f32(1e-16))
            cbase = rows_v * _splat_i32(16)
            for cc in range(16):
                plsc.store_scatter(recip_v, [cbase + _splat_i32(cc)], rv)

        for h in handles:
            h.wait()

        def pool_body(b, carry):
            def sum_body(lo, accs):
                a0, a1 = accs
                for u in range(LU):
                    r = b * L + lo * LU + u
                    a0 = a0 + qrows_v[r, pl.ds(0, 16)]
                    a1 = a1 + qrows_v[r, pl.ds(16, 16)]
                return a0, a1

            acc0, acc1 = lax.fori_loop(
                0, L // LU, sum_body, (_splat_f32(0.0), _splat_f32(0.0)))

            rb = recip_v[pl.ds(b * 16, 16)]
            row = b * FP + F
            srows_v[row, pl.ds(0, 16)] = acc0 * rb
            srows_v[row, pl.ds(16, 16)] = acc1 * rb
            return carry

        lax.fori_loop(0, CB, pool_body, 0)

        pltpu.sync_copy(
            srows_v, out.at[pl.ds(pl.multiple_of(b0 * FP, 32), SPC)])


def kernel(sparse_idx, seq_idx, tables, seq_table):
    sparse_idx = sparse_idx.astype(jnp.int32)
    seq_idx = seq_idx.astype(jnp.int32)

    ftab = tables.reshape(F * VOCAB, D)
    sidx = jnp.concatenate(
        [sparse_idx, jnp.zeros((B, 1), jnp.int32)], axis=1
    ).reshape(NW, NCHUNK * NSG, SG)
    qidx = seq_idx.reshape(-1, QG)
    pidx = jnp.concatenate(
        [seq_idx, jnp.zeros((B, 64 - L), jnp.int32)], axis=1).reshape(-1)

    f = jnp.arange(432, dtype=jnp.int32) % FP
    off = jnp.where(f == F, 0, f * VOCAB).astype(jnp.int32)

    out = _sc_embed(ftab, seq_table, sidx, qidx, pidx, off)
    return out.reshape(B, FP, D)
